# baseline (speedup 1.0000x reference)
import functools

import jax
import jax.numpy as jnp
from jax.experimental import pallas as pl
from jax.experimental.pallas import tpu as pltpu

B, N, C, D, K = 4, 2048, 256, 128, 16
TR = 256
NT = N // TR
_BIG = 3e38
_HI = jax.lax.Precision.HIGHEST


def _dist_tile_t(p, ptq):
    sq_all = jnp.sum(p * p, axis=1, keepdims=True)
    sqq = jnp.sum(ptq * ptq, axis=0, keepdims=True)
    pq = jax.lax.dot_general(p.astype(jnp.bfloat16), ptq.astype(jnp.bfloat16),
                             (((1,), (0,)), ((), ())),
                             preferred_element_type=jnp.float32)
    return jnp.maximum(sq_all + sqq - 2.0 * pq, 0.0)


def _jacobi_normal(a):
    v = [[jnp.full((1, TR), 1.0 if i == j else 0.0, jnp.float32)
          for j in range(3)] for i in range(3)]
    for _ in range(6):
        for (p, q) in ((0, 2), (1, 2), (0, 1)):
            r = 3 - p - q
            app, aqq, apq = a[p][p], a[q][q], a[p][q]
            denom = 2.0 * apq
            tau = (aqq - app) / jnp.where(denom == 0.0, 1.0, denom)
            t = jnp.where(tau >= 0.0, 1.0, -1.0) / (
                jnp.abs(tau) + jnp.sqrt(1.0 + tau * tau))
            t = jnp.where(apq == 0.0, 0.0, t)
            c = jax.lax.rsqrt(1.0 + t * t)
            s = t * c
            new_app = app - t * apq
            new_aqq = aqq + t * apq
            arp, arq = a[r][p], a[r][q]
            new_arp = c * arp - s * arq
            new_arq = s * arp + c * arq
            a[p][p] = new_app
            a[q][q] = new_aqq
            a[p][q] = a[q][p] = jnp.zeros_like(apq)
            a[r][p] = a[p][r] = new_arp
            a[r][q] = a[q][r] = new_arq
            for i in range(3):
                vip, viq = v[i][p], v[i][q]
                v[i][p] = c * vip - s * viq
                v[i][q] = s * vip + c * viq
    l0 = jnp.maximum(a[0][0], 0.0)
    l1 = jnp.maximum(a[1][1], 0.0)
    l2 = jnp.maximum(a[2][2], 0.0)
    pick2 = (l2 <= l1) & (l2 <= l0)
    pick1 = jnp.logical_and(jnp.logical_not(pick2), l1 <= l0)
    def sel(i):
        return jnp.where(pick2, v[i][2], jnp.where(pick1, v[i][1], v[i][0]))
    return sel(0), sel(1), sel(2)


def _geom_kernel(points_ref, pt_ref, ptq_ref, out_ref):
    p = points_ref[0]
    pt = pt_ref[0]
    ptq = ptq_ref[0]

    d2t = _dist_tile_t(p, ptq)
    iota = jax.lax.broadcasted_iota(jnp.int32, (N, TR), 0).astype(jnp.float32)

    avail = jnp.zeros((N, TR), jnp.float32)
    pks = []
    idxs = []
    for _ in range(K):
        masked = d2t + avail
        mn = jnp.min(masked, axis=0, keepdims=True)
        cand = masked == mn
        mi = jnp.min(jnp.where(cand, iota, jnp.float32(N)),
                     axis=0, keepdims=True)
        oh = jnp.logical_and(cand, iota == mi).astype(jnp.float32)
        pks.append(jax.lax.dot_general(pt, oh, (((1,), (0,)), ((), ())),
                                       preferred_element_type=jnp.float32,
                                       precision=_HI))
        idxs.append(mi)
        avail = avail + oh * _BIG

    cen = pks[0]
    for k in range(1, K):
        cen = cen + pks[k]
    cen = cen * jnp.float32(1.0 / K)

    a = [[jnp.zeros((1, TR), jnp.float32) for _ in range(3)] for _ in range(3)]
    for k in range(K):
        dkb = (pks[k] - cen).astype(jnp.bfloat16).astype(jnp.float32)
        d0, d1, d2 = dkb[0:1], dkb[1:2], dkb[2:3]
        a[0][0] = a[0][0] + d0 * d0
        a[1][1] = a[1][1] + d1 * d1
        a[2][2] = a[2][2] + d2 * d2
        a[0][1] = a[0][1] + d0 * d1
        a[0][2] = a[0][2] + d0 * d2
        a[1][2] = a[1][2] + d1 * d2
    a[1][0], a[2][0], a[2][1] = a[0][1], a[0][2], a[1][2]

    nx, ny, nz = _jacobi_normal(a)
    out_ref[0] = jnp.concatenate([nx, ny, nz] + idxs, axis=0)


def _fuse_kernel(nt_ref, ntq_ref, sem_ref,
                 w1_ref, b1_ref, w2_ref, b2_ref, ws_ref, bs_ref,
                 wg_ref, bg_ref, out_ref):
    nt = nt_ref[0]
    ntq = ntq_ref[0]

    iota = jax.lax.broadcasted_iota(jnp.int32, (N, TR), 0).astype(jnp.float32)
    mask = (iota == ntq[3:4]).astype(jnp.float32)
    for k in range(1, K):
        mask = mask + (iota == ntq[3 + k:4 + k]).astype(jnp.float32)

    nrm_t = nt[0:3]
    tt = jax.lax.dot_general(nrm_t, mask, (((1,), (0,)), ((), ())),
                             preferred_element_type=jnp.float32,
                             precision=_HI)
    n_q = ntq[0:3]
    curv = 1.0 - jnp.sum(tt * n_q, axis=0, keepdims=True) * jnp.float32(1.0 / K)

    bf = lambda u: u.astype(jnp.bfloat16)
    def mm(lhs, rhs, dims):
        return jax.lax.dot_general(bf(lhs), bf(rhs), (dims, ((), ())),
                                   preferred_element_type=jnp.float32)

    geo_t = jnp.concatenate([n_q, curv], axis=0)
    h = mm(geo_t, w1_ref[...], ((0,), (0,)))
    h = jnp.maximum(h + b1_ref[...], 0.0)
    ge = mm(h, w2_ref[...], ((1,), (0,))) + b2_ref[...]
    se = mm(sem_ref[0], ws_ref[...], ((1,), (0,))) + bs_ref[...]
    wg = wg_ref[...]
    logits = (mm(ge, wg[:D], ((1,), (0,)))
              + mm(se, wg[D:], ((1,), (0,)))
              + bg_ref[...])
    gate = jax.nn.sigmoid(logits)
    out_ref[0] = gate * se + (1.0 - gate) * ge


@functools.partial(jax.jit, static_argnames=("interpret",))
def kernel(points, semantic_features, W1, b1, W2, b2, Ws, bs, Wg, bg,
           interpret=False):
    pt = jnp.swapaxes(points, 1, 2)

    nt19 = pl.pallas_call(
        _geom_kernel,
        grid=(B, NT),
        in_specs=[
            pl.BlockSpec((1, N, 3), lambda b, i: (b, 0, 0)),
            pl.BlockSpec((1, 3, N), lambda b, i: (b, 0, 0)),
            pl.BlockSpec((1, 3, TR), lambda b, i: (b, 0, i)),
        ],
        out_specs=pl.BlockSpec((1, 19, TR), lambda b, i: (b, 0, i)),
        out_shape=jax.ShapeDtypeStruct((B, 19, N), jnp.float32),
        interpret=interpret,
    )(points, pt, pt)

    full = lambda shape: pl.BlockSpec(shape, lambda b, i: tuple(0 for _ in shape))
    fused = pl.pallas_call(
        _fuse_kernel,
        grid=(B, NT),
        in_specs=[
            pl.BlockSpec((1, 19, N), lambda b, i: (b, 0, 0)),
            pl.BlockSpec((1, 19, TR), lambda b, i: (b, 0, i)),
            pl.BlockSpec((1, TR, C), lambda b, i: (b, i, 0)),
            full((4, 32)), full((1, 32)),
            full((32, D)), full((1, D)),
            full((C, D)), full((1, D)),
            full((2 * D, D)), full((1, D)),
        ],
        out_specs=pl.BlockSpec((1, TR, D), lambda b, i: (b, i, 0)),
        out_shape=jax.ShapeDtypeStruct((B, N, D), jnp.float32),
        interpret=interpret,
    )(nt19, nt19, semantic_features,
      W1, b1.reshape(1, 32), W2, b2.reshape(1, D), Ws, bs.reshape(1, D),
      Wg, bg.reshape(1, D))
    return fused


# speedup vs baseline: 1.4970x; 1.4970x over previous
import functools

import jax
import jax.numpy as jnp
from jax.experimental import pallas as pl
from jax.experimental.pallas import tpu as pltpu

B, N, C, D, K = 4, 2048, 256, 128, 16
TR = 256
NT = N // TR
_BIG = 3e38
_HI = jax.lax.Precision.HIGHEST


def _dist_tile_t(p, ptq):
    sq_all = jnp.sum(p * p, axis=1, keepdims=True)
    sqq = jnp.sum(ptq * ptq, axis=0, keepdims=True)
    pq = jax.lax.dot_general(p.astype(jnp.bfloat16), ptq.astype(jnp.bfloat16),
                             (((1,), (0,)), ((), ())),
                             preferred_element_type=jnp.float32)
    return jnp.maximum(sq_all + sqq - 2.0 * pq, 0.0)


def _jacobi_normal(a):
    v = [[jnp.full((1, TR), 1.0 if i == j else 0.0, jnp.float32)
          for j in range(3)] for i in range(3)]
    for _ in range(6):
        for (p, q) in ((0, 2), (1, 2), (0, 1)):
            r = 3 - p - q
            app, aqq, apq = a[p][p], a[q][q], a[p][q]
            denom = 2.0 * apq
            tau = (aqq - app) / jnp.where(denom == 0.0, 1.0, denom)
            t = jnp.where(tau >= 0.0, 1.0, -1.0) / (
                jnp.abs(tau) + jnp.sqrt(1.0 + tau * tau))
            t = jnp.where(apq == 0.0, 0.0, t)
            c = jax.lax.rsqrt(1.0 + t * t)
            s = t * c
            new_app = app - t * apq
            new_aqq = aqq + t * apq
            arp, arq = a[r][p], a[r][q]
            new_arp = c * arp - s * arq
            new_arq = s * arp + c * arq
            a[p][p] = new_app
            a[q][q] = new_aqq
            a[p][q] = a[q][p] = jnp.zeros_like(apq)
            a[r][p] = a[p][r] = new_arp
            a[r][q] = a[q][r] = new_arq
            for i in range(3):
                vip, viq = v[i][p], v[i][q]
                v[i][p] = c * vip - s * viq
                v[i][q] = s * vip + c * viq
    l0 = jnp.maximum(a[0][0], 0.0)
    l1 = jnp.maximum(a[1][1], 0.0)
    l2 = jnp.maximum(a[2][2], 0.0)
    pick2 = (l2 <= l1) & (l2 <= l0)
    pick1 = jnp.logical_and(jnp.logical_not(pick2), l1 <= l0)
    def sel(i):
        return jnp.where(pick2, v[i][2], jnp.where(pick1, v[i][1], v[i][0]))
    return sel(0), sel(1), sel(2)


def _dezero(d2t):
    iota = jax.lax.broadcasted_iota(jnp.int32, d2t.shape, 0).astype(jnp.float32)
    return jnp.where(d2t == 0.0, iota * 1e-30, d2t)


def _geom_kernel(points_ref, pt_ref, ptq_ref, out_ref):
    p = points_ref[0]
    pt = pt_ref[0]
    ptq = ptq_ref[0]

    d2t = _dezero(_dist_tile_t(p, ptq))

    prev = jnp.full((1, TR), -1.0, jnp.float32)
    pks = []
    for _ in range(K):
        mn = jnp.min(jnp.where(d2t > prev, d2t, _BIG), axis=0, keepdims=True)
        oh = (d2t == mn).astype(jnp.float32)
        pks.append(jax.lax.dot_general(pt, oh, (((1,), (0,)), ((), ())),
                                       preferred_element_type=jnp.float32,
                                       precision=_HI))
        prev = mn
    t = prev

    cen = pks[0]
    for k in range(1, K):
        cen = cen + pks[k]
    cen = cen * jnp.float32(1.0 / K)

    a = [[jnp.zeros((1, TR), jnp.float32) for _ in range(3)] for _ in range(3)]
    for k in range(K):
        dkb = (pks[k] - cen).astype(jnp.bfloat16).astype(jnp.float32)
        d0, d1, d2 = dkb[0:1], dkb[1:2], dkb[2:3]
        a[0][0] = a[0][0] + d0 * d0
        a[1][1] = a[1][1] + d1 * d1
        a[2][2] = a[2][2] + d2 * d2
        a[0][1] = a[0][1] + d0 * d1
        a[0][2] = a[0][2] + d0 * d2
        a[1][2] = a[1][2] + d1 * d2
    a[1][0], a[2][0], a[2][1] = a[0][1], a[0][2], a[1][2]

    nx, ny, nz = _jacobi_normal(a)
    out_ref[0] = jnp.concatenate([nx, ny, nz, t], axis=0)


def _fuse_kernel(points_ref, ptq_ref, nt_ref, ntq_ref, sem_ref,
                 w1_ref, b1_ref, w2_ref, b2_ref, ws_ref, bs_ref,
                 wg_ref, bg_ref, out_ref):
    p = points_ref[0]
    ptq = ptq_ref[0]
    nt = nt_ref[0]
    ntq = ntq_ref[0]

    d2t = _dezero(_dist_tile_t(p, ptq))
    t = ntq[3:4]
    mask = (d2t <= t).astype(jnp.float32)

    nrm_t = nt[0:3]
    tt = jax.lax.dot_general(nrm_t, mask, (((1,), (0,)), ((), ())),
                             preferred_element_type=jnp.float32,
                             precision=_HI)
    n_q = ntq[0:3]
    curv = 1.0 - jnp.sum(tt * n_q, axis=0, keepdims=True) * jnp.float32(1.0 / K)

    bf = lambda u: u.astype(jnp.bfloat16)
    def mm(lhs, rhs, dims):
        return jax.lax.dot_general(bf(lhs), bf(rhs), (dims, ((), ())),
                                   preferred_element_type=jnp.float32)

    geo_t = jnp.concatenate([n_q, curv], axis=0)
    h = mm(geo_t, w1_ref[...], ((0,), (0,)))
    h = jnp.maximum(h + b1_ref[...], 0.0)
    ge = mm(h, w2_ref[...], ((1,), (0,))) + b2_ref[...]
    se = mm(sem_ref[0], ws_ref[...], ((1,), (0,))) + bs_ref[...]
    wg = wg_ref[...]
    logits = (mm(ge, wg[:D], ((1,), (0,)))
              + mm(se, wg[D:], ((1,), (0,)))
              + bg_ref[...])
    gate = jax.nn.sigmoid(logits)
    out_ref[0] = gate * se + (1.0 - gate) * ge


@functools.partial(jax.jit, static_argnames=("interpret",))
def kernel(points, semantic_features, W1, b1, W2, b2, Ws, bs, Wg, bg,
           interpret=False):
    pt = jnp.swapaxes(points, 1, 2)

    nt4 = pl.pallas_call(
        _geom_kernel,
        grid=(B, NT),
        in_specs=[
            pl.BlockSpec((1, N, 3), lambda b, i: (b, 0, 0)),
            pl.BlockSpec((1, 3, N), lambda b, i: (b, 0, 0)),
            pl.BlockSpec((1, 3, TR), lambda b, i: (b, 0, i)),
        ],
        out_specs=pl.BlockSpec((1, 4, TR), lambda b, i: (b, 0, i)),
        out_shape=jax.ShapeDtypeStruct((B, 4, N), jnp.float32),
        interpret=interpret,
    )(points, pt, pt)

    full = lambda shape: pl.BlockSpec(shape, lambda b, i: tuple(0 for _ in shape))
    fused = pl.pallas_call(
        _fuse_kernel,
        grid=(B, NT),
        in_specs=[
            pl.BlockSpec((1, N, 3), lambda b, i: (b, 0, 0)),
            pl.BlockSpec((1, 3, TR), lambda b, i: (b, 0, i)),
            pl.BlockSpec((1, 4, N), lambda b, i: (b, 0, 0)),
            pl.BlockSpec((1, 4, TR), lambda b, i: (b, 0, i)),
            pl.BlockSpec((1, TR, C), lambda b, i: (b, i, 0)),
            full((4, 32)), full((1, 32)),
            full((32, D)), full((1, D)),
            full((C, D)), full((1, D)),
            full((2 * D, D)), full((1, D)),
        ],
        out_specs=pl.BlockSpec((1, TR, D), lambda b, i: (b, i, 0)),
        out_shape=jax.ShapeDtypeStruct((B, N, D), jnp.float32),
        interpret=interpret,
    )(points, pt, nt4, nt4, semantic_features,
      W1, b1.reshape(1, 32), W2, b2.reshape(1, D), Ws, bs.reshape(1, D),
      Wg, bg.reshape(1, D))
    return fused


# speedup vs baseline: 2.5203x; 1.6836x over previous
import functools

import jax
import jax.numpy as jnp
from jax.experimental import pallas as pl
from jax.experimental.pallas import tpu as pltpu

B, N, C, D, K = 4, 2048, 256, 128, 16
TR = 256
NT = N // TR
_BIG = 3e38
_HI = jax.lax.Precision.HIGHEST


def _dist_tile_t(p, ptq):
    sq_all = jnp.sum(p * p, axis=1, keepdims=True)
    sqq = jnp.sum(ptq * ptq, axis=0, keepdims=True)
    pq = jax.lax.dot_general(p.astype(jnp.bfloat16), ptq.astype(jnp.bfloat16),
                             (((1,), (0,)), ((), ())),
                             preferred_element_type=jnp.float32)
    return jnp.maximum(sq_all + sqq - 2.0 * pq, 0.0)


def _jacobi_normal(a):
    v = [[jnp.full((1, TR), 1.0 if i == j else 0.0, jnp.float32)
          for j in range(3)] for i in range(3)]
    for _ in range(6):
        for (p, q) in ((0, 2), (1, 2), (0, 1)):
            r = 3 - p - q
            app, aqq, apq = a[p][p], a[q][q], a[p][q]
            denom = 2.0 * apq
            tau = (aqq - app) / jnp.where(denom == 0.0, 1.0, denom)
            t = jnp.where(tau >= 0.0, 1.0, -1.0) / (
                jnp.abs(tau) + jnp.sqrt(1.0 + tau * tau))
            t = jnp.where(apq == 0.0, 0.0, t)
            c = jax.lax.rsqrt(1.0 + t * t)
            s = t * c
            new_app = app - t * apq
            new_aqq = aqq + t * apq
            arp, arq = a[r][p], a[r][q]
            new_arp = c * arp - s * arq
            new_arq = s * arp + c * arq
            a[p][p] = new_app
            a[q][q] = new_aqq
            a[p][q] = a[q][p] = jnp.zeros_like(apq)
            a[r][p] = a[p][r] = new_arp
            a[r][q] = a[q][r] = new_arq
            for i in range(3):
                vip, viq = v[i][p], v[i][q]
                v[i][p] = c * vip - s * viq
                v[i][q] = s * vip + c * viq
    l0 = jnp.maximum(a[0][0], 0.0)
    l1 = jnp.maximum(a[1][1], 0.0)
    l2 = jnp.maximum(a[2][2], 0.0)
    pick2 = (l2 <= l1) & (l2 <= l0)
    pick1 = jnp.logical_and(jnp.logical_not(pick2), l1 <= l0)
    def sel(i):
        return jnp.where(pick2, v[i][2], jnp.where(pick1, v[i][1], v[i][0]))
    return sel(0), sel(1), sel(2)


def _dezero(d2t):
    iota = jax.lax.broadcasted_iota(jnp.int32, d2t.shape, 0).astype(jnp.float32)
    return jnp.where(d2t == 0.0, iota * 1e-30, d2t)


def _geom_kernel(points_ref, pt_ref, ptq_ref, out_ref):
    p = points_ref[0]
    pt = pt_ref[0]
    ptq = ptq_ref[0]

    d2t = _dezero(_dist_tile_t(p, ptq))

    prev = jnp.full((1, TR), -1.0, jnp.float32)
    for _ in range(K):
        prev = jnp.min(jnp.where(d2t > prev, d2t, _BIG), axis=0, keepdims=True)
    t = prev

    mask = (d2t <= t).astype(jnp.float32)
    s = jax.lax.dot_general(pt, mask, (((1,), (0,)), ((), ())),
                            preferred_element_type=jnp.float32,
                            precision=_HI)
    cen = s * jnp.float32(1.0 / K)

    qs = []
    for c in range(3):
        pc = p[:, c:c + 1]
        q = (pc - cen[c:c + 1]).astype(jnp.bfloat16).astype(jnp.float32)
        qs.append(q * mask)
    a = [[None] * 3 for _ in range(3)]
    for i in range(3):
        for j in range(i, 3):
            a[i][j] = a[j][i] = jnp.sum(qs[i] * qs[j], axis=0, keepdims=True)

    nx, ny, nz = _jacobi_normal(a)
    out_ref[0] = jnp.concatenate([nx, ny, nz, t], axis=0)


def _fuse_kernel(points_ref, ptq_ref, nt_ref, ntq_ref, sem_ref,
                 w1_ref, b1_ref, w2_ref, b2_ref, ws_ref, bs_ref,
                 wg_ref, bg_ref, out_ref):
    p = points_ref[0]
    ptq = ptq_ref[0]
    nt = nt_ref[0]
    ntq = ntq_ref[0]

    d2t = _dezero(_dist_tile_t(p, ptq))
    t = ntq[3:4]
    mask = (d2t <= t).astype(jnp.float32)

    nrm_t = nt[0:3]
    tt = jax.lax.dot_general(nrm_t, mask, (((1,), (0,)), ((), ())),
                             preferred_element_type=jnp.float32,
                             precision=_HI)
    n_q = ntq[0:3]
    curv = 1.0 - jnp.sum(tt * n_q, axis=0, keepdims=True) * jnp.float32(1.0 / K)

    bf = lambda u: u.astype(jnp.bfloat16)
    def mm(lhs, rhs, dims):
        return jax.lax.dot_general(bf(lhs), bf(rhs), (dims, ((), ())),
                                   preferred_element_type=jnp.float32)

    geo_t = jnp.concatenate([n_q, curv], axis=0)
    h = mm(geo_t, w1_ref[...], ((0,), (0,)))
    h = jnp.maximum(h + b1_ref[...], 0.0)
    ge = mm(h, w2_ref[...], ((1,), (0,))) + b2_ref[...]
    se = mm(sem_ref[0], ws_ref[...], ((1,), (0,))) + bs_ref[...]
    wg = wg_ref[...]
    logits = (mm(ge, wg[:D], ((1,), (0,)))
              + mm(se, wg[D:], ((1,), (0,)))
              + bg_ref[...])
    gate = jax.nn.sigmoid(logits)
    out_ref[0] = gate * se + (1.0 - gate) * ge


@functools.partial(jax.jit, static_argnames=("interpret",))
def kernel(points, semantic_features, W1, b1, W2, b2, Ws, bs, Wg, bg,
           interpret=False):
    pt = jnp.swapaxes(points, 1, 2)

    nt4 = pl.pallas_call(
        _geom_kernel,
        grid=(B, NT),
        in_specs=[
            pl.BlockSpec((1, N, 3), lambda b, i: (b, 0, 0)),
            pl.BlockSpec((1, 3, N), lambda b, i: (b, 0, 0)),
            pl.BlockSpec((1, 3, TR), lambda b, i: (b, 0, i)),
        ],
        out_specs=pl.BlockSpec((1, 4, TR), lambda b, i: (b, 0, i)),
        out_shape=jax.ShapeDtypeStruct((B, 4, N), jnp.float32),
        interpret=interpret,
    )(points, pt, pt)

    full = lambda shape: pl.BlockSpec(shape, lambda b, i: tuple(0 for _ in shape))
    fused = pl.pallas_call(
        _fuse_kernel,
        grid=(B, NT),
        in_specs=[
            pl.BlockSpec((1, N, 3), lambda b, i: (b, 0, 0)),
            pl.BlockSpec((1, 3, TR), lambda b, i: (b, 0, i)),
            pl.BlockSpec((1, 4, N), lambda b, i: (b, 0, 0)),
            pl.BlockSpec((1, 4, TR), lambda b, i: (b, 0, i)),
            pl.BlockSpec((1, TR, C), lambda b, i: (b, i, 0)),
            full((4, 32)), full((1, 32)),
            full((32, D)), full((1, D)),
            full((C, D)), full((1, D)),
            full((2 * D, D)), full((1, D)),
        ],
        out_specs=pl.BlockSpec((1, TR, D), lambda b, i: (b, i, 0)),
        out_shape=jax.ShapeDtypeStruct((B, N, D), jnp.float32),
        interpret=interpret,
    )(points, pt, nt4, nt4, semantic_features,
      W1, b1.reshape(1, 32), W2, b2.reshape(1, D), Ws, bs.reshape(1, D),
      Wg, bg.reshape(1, D))
    return fused
